# initial kernel scaffold (unmeasured)
import jax
import jax.numpy as jnp
from jax import lax
from jax.experimental import pallas as pl
from jax.experimental.pallas import tpu as pltpu

T = 4096
D = 2048
V_LOCAL = 8192
HALF = T // 2


def kernel(ids, E):
    my_x = lax.axis_index("x")
    my_y = lax.axis_index("y")

    ids_half = lax.dynamic_slice(ids, (my_y * HALF,), (HALF,))
    local = ids_half - my_x * V_LOCAL
    ok = (local >= 0) & (local < V_LOCAL)
    rows = jnp.take(E, jnp.clip(local, 0, V_LOCAL - 1), axis=0)
    partial = jnp.where(ok[:, None], rows, 0.0)

    def body(partial_ref, out_ref, comm_x, sx, rx, sy, ry):
        x = lax.axis_index("x")
        y = lax.axis_index("y")
        x_nbr = (1 - x, y)
        y_nbr = (x, 1 - y)

        barrier_sem = pltpu.get_barrier_semaphore()
        for nbr in (x_nbr, y_nbr):
            pl.semaphore_signal(
                barrier_sem, inc=1,
                device_id=nbr, device_id_type=pl.DeviceIdType.MESH,
            )
        pl.semaphore_wait(barrier_sem, 2)

        rdma_x = pltpu.make_async_remote_copy(
            src_ref=partial_ref,
            dst_ref=comm_x,
            send_sem=sx,
            recv_sem=rx,
            device_id=x_nbr,
            device_id_type=pl.DeviceIdType.MESH,
        )
        rdma_x.start()
        rdma_x.wait()

        out_ref[pl.ds(y * HALF, HALF), :] = partial_ref[:, :] + comm_x[:, :]

        rdma_y = pltpu.make_async_remote_copy(
            src_ref=out_ref.at[pl.ds(y * HALF, HALF), :],
            dst_ref=out_ref.at[pl.ds(y * HALF, HALF), :],
            send_sem=sy,
            recv_sem=ry,
            device_id=y_nbr,
            device_id_type=pl.DeviceIdType.MESH,
        )
        rdma_y.start()
        rdma_y.wait()

    return pl.pallas_call(
        body,
        out_shape=jax.ShapeDtypeStruct((T, D), jnp.float32),
        in_specs=[pl.BlockSpec(memory_space=pltpu.VMEM)],
        out_specs=pl.BlockSpec(memory_space=pltpu.VMEM),
        scratch_shapes=[
            pltpu.VMEM((HALF, D), jnp.float32),
            pltpu.SemaphoreType.DMA,
            pltpu.SemaphoreType.DMA,
            pltpu.SemaphoreType.DMA,
            pltpu.SemaphoreType.DMA,
        ],
        compiler_params=pltpu.CompilerParams(collective_id=0),
    )(partial)


# baseline (device time: 1912594 ns/iter reference)
import jax
import jax.numpy as jnp
from jax import lax
from jax.experimental import pallas as pl
from jax.experimental.pallas import tpu as pltpu

T = 4096
D = 2048
V_LOCAL = 8192
HALF = T // 2


def kernel(ids, E):
    my_x = lax.axis_index("x")
    my_y = lax.axis_index("y")

    ids_half = lax.dynamic_slice(ids, (my_y * HALF,), (HALF,))
    local = ids_half - my_x * V_LOCAL
    ok = (local >= 0) & (local < V_LOCAL)
    rows = jnp.take(E, jnp.clip(local, 0, V_LOCAL - 1), axis=0)
    partial = jnp.where(ok[:, None], rows, 0.0)

    def body(partial_ref, out_ref, comm_x, red, sx, rx, sy, ry, copy_sem):
        x = lax.axis_index("x")
        y = lax.axis_index("y")
        x_nbr = (1 - x, y)
        y_nbr = (x, 1 - y)

        barrier_sem = pltpu.get_barrier_semaphore()
        for nbr in (x_nbr, y_nbr):
            pl.semaphore_signal(
                barrier_sem, inc=1,
                device_id=nbr, device_id_type=pl.DeviceIdType.MESH,
            )
        pl.semaphore_wait(barrier_sem, 2)

        rdma_x = pltpu.make_async_remote_copy(
            src_ref=partial_ref,
            dst_ref=comm_x,
            send_sem=sx,
            recv_sem=rx,
            device_id=x_nbr,
            device_id_type=pl.DeviceIdType.MESH,
        )
        rdma_x.start()
        rdma_x.wait()

        red[:, :] = partial_ref[:, :] + comm_x[:, :]

        local_copy = pltpu.make_async_copy(
            red, out_ref.at[pl.ds(y * HALF, HALF), :], copy_sem
        )
        local_copy.start()

        rdma_y = pltpu.make_async_remote_copy(
            src_ref=red,
            dst_ref=out_ref.at[pl.ds(y * HALF, HALF), :],
            send_sem=sy,
            recv_sem=ry,
            device_id=y_nbr,
            device_id_type=pl.DeviceIdType.MESH,
        )
        rdma_y.start()
        local_copy.wait()
        rdma_y.wait()

    return pl.pallas_call(
        body,
        out_shape=jax.ShapeDtypeStruct((T, D), jnp.float32),
        in_specs=[pl.BlockSpec(memory_space=pltpu.VMEM)],
        out_specs=pl.BlockSpec(memory_space=pltpu.MemorySpace.HBM),
        scratch_shapes=[
            pltpu.VMEM((HALF, D), jnp.float32),
            pltpu.VMEM((HALF, D), jnp.float32),
            pltpu.SemaphoreType.DMA,
            pltpu.SemaphoreType.DMA,
            pltpu.SemaphoreType.DMA,
            pltpu.SemaphoreType.DMA,
            pltpu.SemaphoreType.DMA,
        ],
        compiler_params=pltpu.CompilerParams(
            collective_id=0,
            vmem_limit_bytes=100 * 1024 * 1024,
        ),
    )(partial)


# device time: 504269 ns/iter; 3.7928x vs baseline; 3.7928x over previous
import jax
import jax.numpy as jnp
from jax import lax
from jax.experimental import pallas as pl
from jax.experimental.pallas import tpu as pltpu

T = 4096
D = 2048
V_LOCAL = 8192
HALF = T // 2
Q = 16


def kernel(ids, E):
    my_x = lax.axis_index("x")
    my_y = lax.axis_index("y")

    ids_half = lax.dynamic_slice(ids, (my_y * HALF,), (HALF,))
    local = ids_half - my_x * V_LOCAL
    ok = (local >= 0) & (local < V_LOCAL)
    lids = jnp.clip(local, 0, V_LOCAL - 1)
    mask = ok.astype(jnp.float32)[:, None]

    def body(lids_ref, mask_ref, e_ref, out_ref,
             raw, snd, comm_x, gsem, sx, rx, sy, ry, copy_sem):
        x = lax.axis_index("x")
        y = lax.axis_index("y")
        x_nbr = (1 - x, y)
        y_nbr = (x, 1 - y)

        barrier_sem = pltpu.get_barrier_semaphore()
        for nbr in (x_nbr, y_nbr):
            pl.semaphore_signal(
                barrier_sem, inc=1,
                device_id=nbr, device_id_type=pl.DeviceIdType.MESH,
            )

        def issue(i):
            pltpu.make_async_copy(
                e_ref.at[pl.ds(lids_ref[i], 1), :],
                raw.at[pl.ds(i, 1), :],
                gsem.at[lax.rem(i, Q)],
            ).start()

        def wait_done(i):
            pltpu.make_async_copy(
                e_ref.at[pl.ds(0, 1), :],
                raw.at[pl.ds(i, 1), :],
                gsem.at[lax.rem(i, Q)],
            ).wait()

        lax.fori_loop(0, Q, lambda i, c: (issue(i), c)[1], 0)

        def steady(i, c):
            wait_done(i - Q)
            issue(i)
            return c

        lax.fori_loop(Q, HALF, steady, 0)
        lax.fori_loop(HALF - Q, HALF, lambda i, c: (wait_done(i), c)[1], 0)

        snd[:, :] = raw[:, :] * mask_ref[:, :]

        pl.semaphore_wait(barrier_sem, 2)

        rdma_x = pltpu.make_async_remote_copy(
            src_ref=snd,
            dst_ref=comm_x,
            send_sem=sx,
            recv_sem=rx,
            device_id=x_nbr,
            device_id_type=pl.DeviceIdType.MESH,
        )
        rdma_x.start()
        rdma_x.wait()

        raw[:, :] = snd[:, :] + comm_x[:, :]

        local_copy = pltpu.make_async_copy(
            raw, out_ref.at[pl.ds(y * HALF, HALF), :], copy_sem
        )
        local_copy.start()

        rdma_y = pltpu.make_async_remote_copy(
            src_ref=raw,
            dst_ref=out_ref.at[pl.ds(y * HALF, HALF), :],
            send_sem=sy,
            recv_sem=ry,
            device_id=y_nbr,
            device_id_type=pl.DeviceIdType.MESH,
        )
        rdma_y.start()
        local_copy.wait()
        rdma_y.wait()

    return pl.pallas_call(
        body,
        out_shape=jax.ShapeDtypeStruct((T, D), jnp.float32),
        in_specs=[
            pl.BlockSpec(memory_space=pltpu.SMEM),
            pl.BlockSpec(memory_space=pltpu.VMEM),
            pl.BlockSpec(memory_space=pltpu.MemorySpace.HBM),
        ],
        out_specs=pl.BlockSpec(memory_space=pltpu.MemorySpace.HBM),
        scratch_shapes=[
            pltpu.VMEM((HALF, D), jnp.float32),
            pltpu.VMEM((HALF, D), jnp.float32),
            pltpu.VMEM((HALF, D), jnp.float32),
            pltpu.SemaphoreType.DMA((Q,)),
            pltpu.SemaphoreType.DMA,
            pltpu.SemaphoreType.DMA,
            pltpu.SemaphoreType.DMA,
            pltpu.SemaphoreType.DMA,
            pltpu.SemaphoreType.DMA,
        ],
        compiler_params=pltpu.CompilerParams(
            collective_id=0,
            vmem_limit_bytes=100 * 1024 * 1024,
        ),
    )(lids, mask, E)


# device time: 250848 ns/iter; 7.6245x vs baseline; 2.0103x over previous
import jax
import jax.numpy as jnp
from jax import lax
from jax.experimental import pallas as pl
from jax.experimental.pallas import tpu as pltpu

T = 4096
D = 2048
V_LOCAL = 8192
HALF = T // 2
Q = 16
C = 8
S = HALF // C


def kernel(ids, E):
    my_x = lax.axis_index("x")
    my_y = lax.axis_index("y")

    ids_half = lax.dynamic_slice(ids, (my_y * HALF,), (HALF,))
    local = ids_half - my_x * V_LOCAL
    ok = (local >= 0) & (local < V_LOCAL)
    lids = jnp.clip(local, 0, V_LOCAL - 1)
    mask = ok.astype(jnp.float32)[:, None]

    def body(lids_ref, mask_ref, e_ref, out_ref,
             raw, snd, comm_x, gsem, sx, rx, sy, ry, copy_sem):
        x = lax.axis_index("x")
        y = lax.axis_index("y")
        x_nbr = (1 - x, y)
        y_nbr = (x, 1 - y)

        barrier_sem = pltpu.get_barrier_semaphore()
        for nbr in (x_nbr, y_nbr):
            pl.semaphore_signal(
                barrier_sem, inc=1,
                device_id=nbr, device_id_type=pl.DeviceIdType.MESH,
            )

        def issue(i):
            pltpu.make_async_copy(
                e_ref.at[pl.ds(lids_ref[i], 1), :],
                raw.at[pl.ds(i, 1), :],
                gsem.at[lax.rem(i, Q)],
            ).start()

        def wait_done(i):
            pltpu.make_async_copy(
                e_ref.at[pl.ds(0, 1), :],
                raw.at[pl.ds(i, 1), :],
                gsem.at[lax.rem(i, Q)],
            ).wait()

        def gather_chunk(c):
            lo, hi = c * S, (c + 1) * S
            lax.fori_loop(lo, lo + Q, lambda i, k: (issue(i), k)[1], 0)
            lax.fori_loop(
                lo + Q, hi, lambda i, k: (wait_done(i - Q), issue(i), k)[2], 0
            )
            lax.fori_loop(hi - Q, hi, lambda i, k: (wait_done(i), k)[1], 0)

        ds = lambda c: pl.ds(c * S, S)

        def make_x(c):
            return pltpu.make_async_remote_copy(
                src_ref=snd.at[ds(c)],
                dst_ref=comm_x.at[ds(c)],
                send_sem=sx.at[c],
                recv_sem=rx.at[c],
                device_id=x_nbr,
                device_id_type=pl.DeviceIdType.MESH,
            )

        def make_y(c):
            return pltpu.make_async_remote_copy(
                src_ref=raw.at[ds(c)],
                dst_ref=out_ref.at[pl.ds(y * HALF + c * S, S), :],
                send_sem=sy.at[c],
                recv_sem=ry.at[c],
                device_id=y_nbr,
                device_id_type=pl.DeviceIdType.MESH,
            )

        def make_local(c):
            return pltpu.make_async_copy(
                raw.at[ds(c)],
                out_ref.at[pl.ds(y * HALF + c * S, S), :],
                copy_sem.at[c],
            )

        def finish_chunk(c):
            make_x(c).wait_recv()
            raw[ds(c), :] = snd[ds(c), :] + comm_x[ds(c), :]
            make_local(c).start()
            make_y(c).start()

        for c in range(C):
            gather_chunk(c)
            snd[ds(c), :] = raw[ds(c), :] * mask_ref[ds(c), :]
            if c == 0:
                pl.semaphore_wait(barrier_sem, 2)
            make_x(c).start()
            if c >= 1:
                finish_chunk(c - 1)
        finish_chunk(C - 1)

        for c in range(C):
            make_x(c).wait_send()
            make_y(c).wait()
            make_local(c).wait()

    return pl.pallas_call(
        body,
        out_shape=jax.ShapeDtypeStruct((T, D), jnp.float32),
        in_specs=[
            pl.BlockSpec(memory_space=pltpu.SMEM),
            pl.BlockSpec(memory_space=pltpu.VMEM),
            pl.BlockSpec(memory_space=pltpu.MemorySpace.HBM),
        ],
        out_specs=pl.BlockSpec(memory_space=pltpu.MemorySpace.HBM),
        scratch_shapes=[
            pltpu.VMEM((HALF, D), jnp.float32),
            pltpu.VMEM((HALF, D), jnp.float32),
            pltpu.VMEM((HALF, D), jnp.float32),
            pltpu.SemaphoreType.DMA((Q,)),
            pltpu.SemaphoreType.DMA((C,)),
            pltpu.SemaphoreType.DMA((C,)),
            pltpu.SemaphoreType.DMA((C,)),
            pltpu.SemaphoreType.DMA((C,)),
            pltpu.SemaphoreType.DMA((C,)),
        ],
        compiler_params=pltpu.CompilerParams(
            collective_id=0,
            vmem_limit_bytes=100 * 1024 * 1024,
        ),
    )(lids, mask, E)


# device time: 245409 ns/iter; 7.7935x vs baseline; 1.0222x over previous
import jax
import jax.numpy as jnp
from jax import lax
from jax.experimental import pallas as pl
from jax.experimental.pallas import tpu as pltpu

T = 4096
D = 2048
V_LOCAL = 8192
HALF = T // 2
Q = 32
C = 8
S = HALF // C


def kernel(ids, E):
    my_x = lax.axis_index("x")
    my_y = lax.axis_index("y")

    ids_half = lax.dynamic_slice(ids, (my_y * HALF,), (HALF,))
    local = ids_half - my_x * V_LOCAL
    ok = (local >= 0) & (local < V_LOCAL)
    lids = jnp.clip(local, 0, V_LOCAL - 1)
    mask = ok.astype(jnp.float32)[:, None]

    def body(lids_ref, mask_ref, e_ref, out_ref,
             raw, snd, comm_x, gsem, sx, rx, sy, ry, copy_sem):
        x = lax.axis_index("x")
        y = lax.axis_index("y")
        x_nbr = (1 - x, y)
        y_nbr = (x, 1 - y)

        barrier_sem = pltpu.get_barrier_semaphore()
        for nbr in (x_nbr, y_nbr):
            pl.semaphore_signal(
                barrier_sem, inc=1,
                device_id=nbr, device_id_type=pl.DeviceIdType.MESH,
            )

        def issue(i):
            pltpu.make_async_copy(
                e_ref.at[pl.ds(lids_ref[i], 1), :],
                raw.at[pl.ds(i, 1), :],
                gsem.at[lax.rem(i, Q)],
            ).start()

        def wait_done(i):
            pltpu.make_async_copy(
                e_ref.at[pl.ds(0, 1), :],
                raw.at[pl.ds(i, 1), :],
                gsem.at[lax.rem(i, Q)],
            ).wait()

        def gather_chunk(c):
            lo, hi = c * S, (c + 1) * S
            if c == 0:
                lax.fori_loop(0, Q, lambda i, k: (issue(i), k)[1], 0)
            if c < C - 1:
                lax.fori_loop(
                    lo, hi, lambda i, k: (wait_done(i), issue(i + Q), k)[2], 0
                )
            else:
                lax.fori_loop(
                    lo, hi - Q,
                    lambda i, k: (wait_done(i), issue(i + Q), k)[2], 0,
                )
                lax.fori_loop(hi - Q, hi, lambda i, k: (wait_done(i), k)[1], 0)

        ds = lambda c: pl.ds(c * S, S)

        def make_x(c):
            return pltpu.make_async_remote_copy(
                src_ref=snd.at[ds(c)],
                dst_ref=comm_x.at[ds(c)],
                send_sem=sx.at[c],
                recv_sem=rx.at[c],
                device_id=x_nbr,
                device_id_type=pl.DeviceIdType.MESH,
            )

        def make_y(c):
            return pltpu.make_async_remote_copy(
                src_ref=raw.at[ds(c)],
                dst_ref=out_ref.at[pl.ds(y * HALF + c * S, S), :],
                send_sem=sy.at[c],
                recv_sem=ry.at[c],
                device_id=y_nbr,
                device_id_type=pl.DeviceIdType.MESH,
            )

        def make_local(c):
            return pltpu.make_async_copy(
                raw.at[ds(c)],
                out_ref.at[pl.ds(y * HALF + c * S, S), :],
                copy_sem.at[c],
            )

        def finish_chunk(c):
            make_x(c).wait_recv()
            raw[ds(c), :] = snd[ds(c), :] + comm_x[ds(c), :]
            make_local(c).start()
            make_y(c).start()

        for c in range(C):
            gather_chunk(c)
            snd[ds(c), :] = raw[ds(c), :] * mask_ref[ds(c), :]
            if c == 0:
                pl.semaphore_wait(barrier_sem, 2)
            make_x(c).start()
            if c >= 1:
                finish_chunk(c - 1)
        finish_chunk(C - 1)

        for c in range(C):
            make_x(c).wait_send()
            make_y(c).wait()
            make_local(c).wait()

    return pl.pallas_call(
        body,
        out_shape=jax.ShapeDtypeStruct((T, D), jnp.float32),
        in_specs=[
            pl.BlockSpec(memory_space=pltpu.SMEM),
            pl.BlockSpec(memory_space=pltpu.VMEM),
            pl.BlockSpec(memory_space=pltpu.MemorySpace.HBM),
        ],
        out_specs=pl.BlockSpec(memory_space=pltpu.MemorySpace.HBM),
        scratch_shapes=[
            pltpu.VMEM((HALF, D), jnp.float32),
            pltpu.VMEM((HALF, D), jnp.float32),
            pltpu.VMEM((HALF, D), jnp.float32),
            pltpu.SemaphoreType.DMA((Q,)),
            pltpu.SemaphoreType.DMA((C,)),
            pltpu.SemaphoreType.DMA((C,)),
            pltpu.SemaphoreType.DMA((C,)),
            pltpu.SemaphoreType.DMA((C,)),
            pltpu.SemaphoreType.DMA((C,)),
        ],
        compiler_params=pltpu.CompilerParams(
            collective_id=0,
            vmem_limit_bytes=100 * 1024 * 1024,
        ),
    )(lids, mask, E)


# device time: 241952 ns/iter; 7.9048x vs baseline; 1.0143x over previous
import jax
import jax.numpy as jnp
from jax import lax
from jax.experimental import pallas as pl
from jax.experimental.pallas import tpu as pltpu

T = 4096
D = 2048
V_LOCAL = 8192
HALF = T // 2
Q = 32
C = 8
S = HALF // C


def kernel(ids, E):
    my_x = lax.axis_index("x")
    my_y = lax.axis_index("y")

    ids_half = lax.dynamic_slice(ids, (my_y * HALF,), (HALF,))
    local = ids_half - my_x * V_LOCAL
    ok = (local >= 0) & (local < V_LOCAL)
    lids = jnp.where(ok, local, -1)

    def body(lids_ref, e_ref, out_ref,
             raw, snd, zrow, comm_x, gsem, sx, rx, sy, ry, copy_sem):
        x = lax.axis_index("x")
        y = lax.axis_index("y")
        x_nbr = (1 - x, y)
        y_nbr = (x, 1 - y)

        barrier_sem = pltpu.get_barrier_semaphore()
        for nbr in (x_nbr, y_nbr):
            pl.semaphore_signal(
                barrier_sem, inc=1,
                device_id=nbr, device_id_type=pl.DeviceIdType.MESH,
            )

        zrow[:, :] = jnp.zeros_like(zrow)

        def issue(i):
            idx = lids_ref[i]

            @pl.when(idx >= 0)
            def _():
                pltpu.make_async_copy(
                    e_ref.at[pl.ds(idx, 1), :],
                    snd.at[pl.ds(i, 1), :],
                    gsem.at[lax.rem(i, Q)],
                ).start()

            @pl.when(idx < 0)
            def _():
                pltpu.make_async_copy(
                    zrow,
                    snd.at[pl.ds(i, 1), :],
                    gsem.at[lax.rem(i, Q)],
                ).start()

        def wait_done(i):
            pltpu.make_async_copy(
                e_ref.at[pl.ds(0, 1), :],
                snd.at[pl.ds(i, 1), :],
                gsem.at[lax.rem(i, Q)],
            ).wait()

        def gather_chunk(c):
            lo, hi = c * S, (c + 1) * S
            if c == 0:
                lax.fori_loop(0, Q, lambda i, k: (issue(i), k)[1], 0,
                              unroll=8)
            if c < C - 1:
                lax.fori_loop(
                    lo, hi, lambda i, k: (wait_done(i), issue(i + Q), k)[2], 0,
                    unroll=8,
                )
            else:
                lax.fori_loop(
                    lo, hi - Q,
                    lambda i, k: (wait_done(i), issue(i + Q), k)[2], 0,
                    unroll=8,
                )
                lax.fori_loop(hi - Q, hi, lambda i, k: (wait_done(i), k)[1], 0,
                              unroll=8)

        ds = lambda c: pl.ds(c * S, S)

        def make_x(c):
            return pltpu.make_async_remote_copy(
                src_ref=snd.at[ds(c)],
                dst_ref=comm_x.at[ds(c)],
                send_sem=sx.at[c],
                recv_sem=rx.at[c],
                device_id=x_nbr,
                device_id_type=pl.DeviceIdType.MESH,
            )

        def make_y(c):
            return pltpu.make_async_remote_copy(
                src_ref=raw.at[ds(c)],
                dst_ref=out_ref.at[pl.ds(y * HALF + c * S, S), :],
                send_sem=sy.at[c],
                recv_sem=ry.at[c],
                device_id=y_nbr,
                device_id_type=pl.DeviceIdType.MESH,
            )

        def make_local(c):
            return pltpu.make_async_copy(
                raw.at[ds(c)],
                out_ref.at[pl.ds(y * HALF + c * S, S), :],
                copy_sem.at[c],
            )

        def finish_chunk(c):
            make_x(c).wait_recv()
            raw[ds(c), :] = snd[ds(c), :] + comm_x[ds(c), :]
            make_local(c).start()
            make_y(c).start()

        for c in range(C):
            gather_chunk(c)
            if c == 0:
                pl.semaphore_wait(barrier_sem, 2)
            make_x(c).start()
            if c >= 1:
                finish_chunk(c - 1)
        finish_chunk(C - 1)

        for c in range(C):
            make_x(c).wait_send()
            make_y(c).wait()
            make_local(c).wait()

    return pl.pallas_call(
        body,
        out_shape=jax.ShapeDtypeStruct((T, D), jnp.float32),
        in_specs=[
            pl.BlockSpec(memory_space=pltpu.SMEM),
            pl.BlockSpec(memory_space=pltpu.MemorySpace.HBM),
        ],
        out_specs=pl.BlockSpec(memory_space=pltpu.MemorySpace.HBM),
        scratch_shapes=[
            pltpu.VMEM((HALF, D), jnp.float32),
            pltpu.VMEM((HALF, D), jnp.float32),
            pltpu.VMEM((1, D), jnp.float32),
            pltpu.VMEM((HALF, D), jnp.float32),
            pltpu.SemaphoreType.DMA((Q,)),
            pltpu.SemaphoreType.DMA((C,)),
            pltpu.SemaphoreType.DMA((C,)),
            pltpu.SemaphoreType.DMA((C,)),
            pltpu.SemaphoreType.DMA((C,)),
            pltpu.SemaphoreType.DMA((C,)),
        ],
        compiler_params=pltpu.CompilerParams(
            collective_id=0,
            vmem_limit_bytes=100 * 1024 * 1024,
        ),
    )(lids, E)


# device time: 227448 ns/iter; 8.4089x vs baseline; 1.0638x over previous
import jax
import jax.numpy as jnp
from jax import lax
from jax.experimental import pallas as pl
from jax.experimental.pallas import tpu as pltpu

T = 4096
D = 2048
V_LOCAL = 8192
HALF = T // 2
Q = 32
C = 16
S = HALF // C


def kernel(ids, E):
    my_x = lax.axis_index("x")
    my_y = lax.axis_index("y")

    ids_half = lax.dynamic_slice(ids, (my_y * HALF,), (HALF,))
    local = ids_half - my_x * V_LOCAL
    ok = (local >= 0) & (local < V_LOCAL)
    lids = jnp.where(ok, local, -1)

    def body(lids_ref, e_ref, out_ref,
             raw, snd, zrow, comm_x, gsem, sx, rx, sy, ry, copy_sem):
        x = lax.axis_index("x")
        y = lax.axis_index("y")
        x_nbr = (1 - x, y)
        y_nbr = (x, 1 - y)

        barrier_sem = pltpu.get_barrier_semaphore()
        for nbr in (x_nbr, y_nbr):
            pl.semaphore_signal(
                barrier_sem, inc=1,
                device_id=nbr, device_id_type=pl.DeviceIdType.MESH,
            )

        zrow[:, :] = jnp.zeros_like(zrow)

        def issue(i):
            idx = lids_ref[i]

            @pl.when(idx >= 0)
            def _():
                pltpu.make_async_copy(
                    e_ref.at[pl.ds(idx, 1), :],
                    snd.at[pl.ds(i, 1), :],
                    gsem.at[lax.rem(i, Q)],
                ).start()

            @pl.when(idx < 0)
            def _():
                pltpu.make_async_copy(
                    zrow,
                    snd.at[pl.ds(i, 1), :],
                    gsem.at[lax.rem(i, Q)],
                ).start()

        def wait_done(i):
            pltpu.make_async_copy(
                e_ref.at[pl.ds(0, 1), :],
                snd.at[pl.ds(i, 1), :],
                gsem.at[lax.rem(i, Q)],
            ).wait()

        def gather_chunk(c):
            lo, hi = c * S, (c + 1) * S
            if c == 0:
                lax.fori_loop(0, Q, lambda i, k: (issue(i), k)[1], 0,
                              unroll=8)
            if c < C - 1:
                lax.fori_loop(
                    lo, hi, lambda i, k: (wait_done(i), issue(i + Q), k)[2], 0,
                    unroll=8,
                )
            else:
                lax.fori_loop(
                    lo, hi - Q,
                    lambda i, k: (wait_done(i), issue(i + Q), k)[2], 0,
                    unroll=8,
                )
                lax.fori_loop(hi - Q, hi, lambda i, k: (wait_done(i), k)[1], 0,
                              unroll=8)

        ds = lambda c: pl.ds(c * S, S)

        def make_x(c):
            return pltpu.make_async_remote_copy(
                src_ref=snd.at[ds(c)],
                dst_ref=comm_x.at[ds(c)],
                send_sem=sx.at[c],
                recv_sem=rx.at[c],
                device_id=x_nbr,
                device_id_type=pl.DeviceIdType.MESH,
            )

        def make_y(c):
            return pltpu.make_async_remote_copy(
                src_ref=raw.at[ds(c)],
                dst_ref=out_ref.at[pl.ds(y * HALF + c * S, S), :],
                send_sem=sy.at[c],
                recv_sem=ry.at[c],
                device_id=y_nbr,
                device_id_type=pl.DeviceIdType.MESH,
            )

        def make_local(c):
            return pltpu.make_async_copy(
                raw.at[ds(c)],
                out_ref.at[pl.ds(y * HALF + c * S, S), :],
                copy_sem.at[c],
            )

        def finish_chunk(c):
            make_x(c).wait_recv()
            raw[ds(c), :] = snd[ds(c), :] + comm_x[ds(c), :]
            make_local(c).start()
            make_y(c).start()

        for c in range(C):
            gather_chunk(c)
            if c == 0:
                pl.semaphore_wait(barrier_sem, 2)
            make_x(c).start()
            if c >= 1:
                finish_chunk(c - 1)
        finish_chunk(C - 1)

        for c in range(C):
            make_x(c).wait_send()
            make_y(c).wait()
            make_local(c).wait()

    return pl.pallas_call(
        body,
        out_shape=jax.ShapeDtypeStruct((T, D), jnp.float32),
        in_specs=[
            pl.BlockSpec(memory_space=pltpu.SMEM),
            pl.BlockSpec(memory_space=pltpu.MemorySpace.HBM),
        ],
        out_specs=pl.BlockSpec(memory_space=pltpu.MemorySpace.HBM),
        scratch_shapes=[
            pltpu.VMEM((HALF, D), jnp.float32),
            pltpu.VMEM((HALF, D), jnp.float32),
            pltpu.VMEM((1, D), jnp.float32),
            pltpu.VMEM((HALF, D), jnp.float32),
            pltpu.SemaphoreType.DMA((Q,)),
            pltpu.SemaphoreType.DMA((C,)),
            pltpu.SemaphoreType.DMA((C,)),
            pltpu.SemaphoreType.DMA((C,)),
            pltpu.SemaphoreType.DMA((C,)),
            pltpu.SemaphoreType.DMA((C,)),
        ],
        compiler_params=pltpu.CompilerParams(
            collective_id=0,
            vmem_limit_bytes=100 * 1024 * 1024,
        ),
    )(lids, E)


# device time: 134034 ns/iter; 14.2695x vs baseline; 1.6969x over previous
import jax
import jax.numpy as jnp
from jax import lax
from jax.experimental import pallas as pl
from jax.experimental.pallas import tpu as pltpu

T = 4096
D = 2048
V_LOCAL = 8192
HALF = T // 2
Q = 32
C = 16
S = HALF // C
YLAG = 2


def kernel(ids, E):
    my_x = lax.axis_index("x")
    my_y = lax.axis_index("y")

    ids_half = lax.dynamic_slice(ids, (my_y * HALF,), (HALF,))
    local = ids_half - my_x * V_LOCAL
    ok = (local >= 0) & (local < V_LOCAL)
    lids = jnp.where(ok, local, -1)

    def body(lids_ref, e_ref, out_ref,
             g, xs, cx, ys, cy, stage, zrow,
             gsem, sx, rx, sy, ry, copy_sem, ssem):
        x = lax.axis_index("x")
        y = lax.axis_index("y")
        x_nbr = (1 - x, y)
        y_nbr = (x, 1 - y)

        barrier_sem = pltpu.get_barrier_semaphore()
        for nbr in (x_nbr, y_nbr):
            pl.semaphore_signal(
                barrier_sem, inc=1,
                device_id=nbr, device_id_type=pl.DeviceIdType.MESH,
            )

        zrow[:, :] = jnp.zeros_like(zrow)

        def issue(i):
            idx = lids_ref[i]

            @pl.when(idx >= 0)
            def _():
                pltpu.make_async_copy(
                    e_ref.at[pl.ds(idx, 1), :],
                    g.at[pl.ds(i, 1), :],
                    gsem.at[lax.rem(i, Q)],
                ).start()

            @pl.when(idx < 0)
            def _():
                pltpu.make_async_copy(
                    zrow,
                    g.at[pl.ds(i, 1), :],
                    gsem.at[lax.rem(i, Q)],
                ).start()

        def wait_done(i):
            pltpu.make_async_copy(
                e_ref.at[pl.ds(0, 1), :],
                g.at[pl.ds(i, 1), :],
                gsem.at[lax.rem(i, Q)],
            ).wait()

        def gather_chunk(c):
            lo, hi = c * S, (c + 1) * S
            if c == 0:
                lax.fori_loop(0, Q, lambda i, k: (issue(i), k)[1], 0,
                              unroll=8)
            if c < C - 1:
                lax.fori_loop(
                    lo, hi, lambda i, k: (wait_done(i), issue(i + Q), k)[2], 0,
                    unroll=8,
                )
            else:
                lax.fori_loop(
                    lo, hi - Q,
                    lambda i, k: (wait_done(i), issue(i + Q), k)[2], 0,
                    unroll=8,
                )
                lax.fori_loop(hi - Q, hi, lambda i, k: (wait_done(i), k)[1], 0,
                              unroll=8)

        ds = lambda c: pl.ds(c * S, S)

        def make_x(c):
            return pltpu.make_async_remote_copy(
                src_ref=xs.at[ds(c)],
                dst_ref=cx.at[ds(c)],
                send_sem=sx.at[c],
                recv_sem=rx.at[c],
                device_id=x_nbr,
                device_id_type=pl.DeviceIdType.MESH,
            )

        def make_y(c):
            return pltpu.make_async_remote_copy(
                src_ref=ys.at[ds(c)],
                dst_ref=cy.at[ds(c)],
                send_sem=sy.at[c],
                recv_sem=ry.at[c],
                device_id=y_nbr,
                device_id_type=pl.DeviceIdType.MESH,
            )

        def make_local(c):
            return pltpu.make_async_copy(
                g.at[ds(c)],
                out_ref.at[pl.ds(y * HALF + c * S, S), :],
                copy_sem.at[c],
            )

        def make_stage(c):
            return pltpu.make_async_copy(
                stage.at[c % 2],
                out_ref.at[pl.ds((1 - y) * HALF + c * S, S), :],
                ssem.at[c % 2],
            )

        def finish_x(c):
            make_x(c).wait_recv()
            g[ds(c), :] = g[ds(c), :] + cx[ds(c), :].astype(jnp.float32)
            make_local(c).start()
            ys[ds(c), :] = g[ds(c), :].astype(jnp.bfloat16)
            make_y(c).start()

        def finish_y(c):
            make_y(c).wait_recv()
            if c >= 2:
                make_stage(c - 2).wait()
            stage[c % 2, :, :] = cy[ds(c), :].astype(jnp.float32)
            make_stage(c).start()

        for c in range(C):
            gather_chunk(c)
            xs[ds(c), :] = g[ds(c), :].astype(jnp.bfloat16)
            if c == 0:
                pl.semaphore_wait(barrier_sem, 2)
            make_x(c).start()
            if c >= 1:
                finish_x(c - 1)
            if c >= 1 + YLAG:
                finish_y(c - 1 - YLAG)
        finish_x(C - 1)
        for c in range(C - YLAG - 1, C):
            finish_y(c)

        for c in range(C):
            make_x(c).wait_send()
            make_y(c).wait_send()
            make_local(c).wait()
        make_stage(C - 2).wait()
        make_stage(C - 1).wait()

    return pl.pallas_call(
        body,
        out_shape=jax.ShapeDtypeStruct((T, D), jnp.float32),
        in_specs=[
            pl.BlockSpec(memory_space=pltpu.SMEM),
            pl.BlockSpec(memory_space=pltpu.MemorySpace.HBM),
        ],
        out_specs=pl.BlockSpec(memory_space=pltpu.MemorySpace.HBM),
        scratch_shapes=[
            pltpu.VMEM((HALF, D), jnp.float32),
            pltpu.VMEM((HALF, D), jnp.bfloat16),
            pltpu.VMEM((HALF, D), jnp.bfloat16),
            pltpu.VMEM((HALF, D), jnp.bfloat16),
            pltpu.VMEM((HALF, D), jnp.bfloat16),
            pltpu.VMEM((2, S, D), jnp.float32),
            pltpu.VMEM((1, D), jnp.float32),
            pltpu.SemaphoreType.DMA((Q,)),
            pltpu.SemaphoreType.DMA((C,)),
            pltpu.SemaphoreType.DMA((C,)),
            pltpu.SemaphoreType.DMA((C,)),
            pltpu.SemaphoreType.DMA((C,)),
            pltpu.SemaphoreType.DMA((C,)),
            pltpu.SemaphoreType.DMA((2,)),
        ],
        compiler_params=pltpu.CompilerParams(
            collective_id=0,
            vmem_limit_bytes=100 * 1024 * 1024,
        ),
    )(lids, E)
